# baseline (speedup 1.0000x reference)
import functools

import jax
import jax.numpy as jnp
from jax import lax
from jax.experimental import pallas as pl
from jax.experimental.pallas import tpu as pltpu
from jax.experimental.pallas import tpu_sc as plsc

D_MODEL = 1024
NUM_TYPES = 8
B_TOTAL = 4 * 8192

NUM_CORES = 2
NUM_SUBCORES = 16
NUM_WORKERS = NUM_CORES * NUM_SUBCORES
B_PER_W = B_TOTAL // NUM_WORKERS
CHUNK = 32
N_CHUNKS = B_PER_W // CHUNK
N_PAIRS = N_CHUNKS // 2
N_SUB = 4
SUB = CHUNK // N_SUB


@functools.partial(
    pl.kernel,
    mesh=plsc.VectorSubcoreMesh(core_axis_name="c", subcore_axis_name="s"),
    out_type=jax.ShapeDtypeStruct((B_TOTAL, D_MODEL), jnp.float32),
    scratch_types=[
        pltpu.VMEM((B_PER_W,), jnp.int32),
        pltpu.VMEM((CHUNK, D_MODEL), jnp.float32),
        pltpu.VMEM((CHUNK, D_MODEL), jnp.float32),
        pltpu.SemaphoreType.DMA((2, N_SUB)),
        pltpu.SemaphoreType.DMA((2,)),
    ],
)
def _emb_lookup(idx_hbm, table_hbm, out_hbm, idx_v, buf0, buf1, gsem, ssem):
    wid = lax.axis_index("s") * NUM_CORES + lax.axis_index("c")
    base = wid * B_PER_W
    pltpu.sync_copy(idx_hbm.at[pl.ds(base, B_PER_W)], idx_v)

    bufs = (buf0, buf1)

    def start_gather(i, b):
        for k in range(N_SUB):
            pltpu.async_copy(
                table_hbm.at[idx_v.at[pl.ds(i * CHUNK + k * SUB, SUB)]],
                bufs[b].at[pl.ds(k * SUB, SUB)],
                gsem.at[b, k],
            )

    def wait_gather(b):
        for k in range(N_SUB):
            pltpu.make_async_copy(
                table_hbm.at[idx_v.at[pl.ds(0, SUB)]],
                bufs[b].at[pl.ds(k * SUB, SUB)],
                gsem.at[b, k],
            ).wait()

    def start_store(i, b):
        pltpu.async_copy(
            bufs[b], out_hbm.at[pl.ds(base + i * CHUNK, CHUNK)], ssem.at[b]
        )

    def wait_store(b):
        pltpu.make_async_copy(
            bufs[b], out_hbm.at[pl.ds(base, CHUNK)], ssem.at[b]
        ).wait()

    start_gather(0, 0)
    start_gather(1, 1)
    wait_gather(0)
    start_store(0, 0)

    def body(j, carry):
        for b in range(2):
            i = 2 * j + b
            wait_store(b)
            start_gather(i, b)
            wait_gather(1 - b)
            start_store(i - 1, 1 - b)
        return carry

    lax.fori_loop(1, N_PAIRS, body, 0, unroll=True)

    wait_gather(1)
    start_store(N_CHUNKS - 1, 1)
    wait_store(0)
    wait_store(1)


def kernel(token_types, type_embedding_weight):
    flat_idx = token_types.reshape(B_TOTAL).astype(jnp.int32)
    out = _emb_lookup(flat_idx, type_embedding_weight)
    return out.reshape(token_types.shape + (D_MODEL,))


# speedup vs baseline: 1.1345x; 1.1345x over previous
import functools

import jax
import jax.numpy as jnp
from jax import lax
from jax.experimental import pallas as pl
from jax.experimental.pallas import tpu as pltpu
from jax.experimental.pallas import tpu_sc as plsc

D_MODEL = 1024
NUM_TYPES = 8
B_TOTAL = 4 * 8192

NUM_CORES = 2
NUM_SUBCORES = 16
NUM_WORKERS = NUM_CORES * NUM_SUBCORES
B_PER_W = B_TOTAL // NUM_WORKERS
LANES = 16
CHUNK = 16
N_CHUNKS = B_PER_W // CHUNK
N_PAIRS = N_CHUNKS // 2
COL_STEPS = D_MODEL // LANES


@functools.partial(
    pl.kernel,
    mesh=plsc.VectorSubcoreMesh(core_axis_name="c", subcore_axis_name="s"),
    out_type=jax.ShapeDtypeStruct((B_TOTAL, D_MODEL), jnp.float32),
    scratch_types=[
        pltpu.VMEM((B_PER_W,), jnp.int32),
        pltpu.VMEM((NUM_TYPES, D_MODEL), jnp.float32),
        pltpu.VMEM((CHUNK, D_MODEL), jnp.float32),
        pltpu.VMEM((CHUNK, D_MODEL), jnp.float32),
        pltpu.SemaphoreType.DMA,
        pltpu.SemaphoreType.DMA,
    ],
)
def _emb_lookup(idx_hbm, table_hbm, out_hbm, idx_v, table_v, buf0, buf1, s0, s1):
    wid = lax.axis_index("s") * NUM_CORES + lax.axis_index("c")
    base = wid * B_PER_W
    pltpu.sync_copy(idx_hbm.at[pl.ds(base, B_PER_W)], idx_v)
    pltpu.sync_copy(table_hbm, table_v)

    bufs = (buf0, buf1)
    ssems = (s0, s1)

    def fill_chunk(i, b):
        buf = bufs[b]
        vec = idx_v[pl.ds(i * CHUNK, CHUNK)]
        for j in range(CHUNK):
            t = vec[j]
            row = table_v.at[t]
            dst = buf.at[j]

            def col_body(u, carry):
                c = u * 16 * LANES
                for k in range(16):
                    dst[pl.ds(c + k * LANES, LANES)] = row[pl.ds(c + k * LANES, LANES)]
                return carry

            lax.fori_loop(0, COL_STEPS // 16, col_body, 0)

    def start_store(i, b):
        pltpu.async_copy(
            bufs[b], out_hbm.at[pl.ds(base + i * CHUNK, CHUNK)], ssems[b]
        )

    def wait_store(b):
        pltpu.make_async_copy(
            bufs[b], out_hbm.at[pl.ds(base, CHUNK)], ssems[b]
        ).wait()

    def body(p, carry):
        for b in range(2):
            i = 2 * p + b

            @pl.when(p >= 1)
            def _():
                wait_store(b)

            fill_chunk(i, b)
            start_store(i, b)
        return carry

    lax.fori_loop(0, N_PAIRS, body, 0)

    wait_store(0)
    wait_store(1)


def kernel(token_types, type_embedding_weight):
    flat_idx = token_types.reshape(B_TOTAL).astype(jnp.int32)
    out = _emb_lookup(flat_idx, type_embedding_weight)
    return out.reshape(token_types.shape + (D_MODEL,))


# speedup vs baseline: 4.8223x; 4.2506x over previous
import functools

import jax
import jax.numpy as jnp
from jax import lax
from jax.experimental import pallas as pl
from jax.experimental.pallas import tpu as pltpu
from jax.experimental.pallas import tpu_sc as plsc

D_MODEL = 1024
NUM_TYPES = 8
B_TOTAL = 4 * 8192

NUM_CORES = 2
NUM_SUBCORES = 16
NUM_WORKERS = NUM_CORES * NUM_SUBCORES
B_PER_W = B_TOTAL // NUM_WORKERS
LANES = 16
CHUNK = 16
N_CHUNKS = B_PER_W // CHUNK
N_PAIRS = N_CHUNKS // 2
COL_STEPS = D_MODEL // LANES


@functools.partial(
    pl.kernel,
    mesh=plsc.VectorSubcoreMesh(core_axis_name="c", subcore_axis_name="s"),
    out_type=jax.ShapeDtypeStruct((B_TOTAL, D_MODEL), jnp.float32),
    scratch_types=[
        pltpu.VMEM((B_PER_W,), jnp.int32),
        pltpu.VMEM((NUM_TYPES, D_MODEL), jnp.float32),
        pltpu.VMEM((CHUNK, D_MODEL), jnp.float32),
        pltpu.VMEM((CHUNK, D_MODEL), jnp.float32),
        pltpu.SemaphoreType.DMA,
        pltpu.SemaphoreType.DMA,
    ],
)
def _emb_lookup(idx_hbm, table_hbm, out_hbm, idx_v, table_v, buf0, buf1, s0, s1):
    wid = lax.axis_index("s") * NUM_CORES + lax.axis_index("c")
    base = wid * B_PER_W
    pltpu.sync_copy(idx_hbm.at[pl.ds(base, B_PER_W)], idx_v)
    pltpu.sync_copy(table_hbm, table_v)

    bufs = (buf0, buf1)
    ssems = (s0, s1)

    def fill_chunk(i, b):
        buf = bufs[b]
        vec = idx_v[pl.ds(i * CHUNK, CHUNK)]
        for j in range(CHUNK):
            t = vec[j]
            row = table_v.at[t]
            dst = buf.at[j]

            @plsc.parallel_loop(0, COL_STEPS, unroll=8)
            def col_body(u):
                c = u * LANES
                dst[pl.ds(c, LANES)] = row[pl.ds(c, LANES)]

    def start_store(i, b):
        pltpu.async_copy(
            bufs[b], out_hbm.at[pl.ds(base + i * CHUNK, CHUNK)], ssems[b]
        )

    def wait_store(b):
        pltpu.make_async_copy(
            bufs[b], out_hbm.at[pl.ds(base, CHUNK)], ssems[b]
        ).wait()

    def body(p, carry):
        for b in range(2):
            i = 2 * p + b

            @pl.when(p >= 1)
            def _():
                wait_store(b)

            fill_chunk(i, b)
            start_store(i, b)
        return carry

    lax.fori_loop(0, N_PAIRS, body, 0)

    wait_store(0)
    wait_store(1)


def kernel(token_types, type_embedding_weight):
    flat_idx = token_types.reshape(B_TOTAL).astype(jnp.int32)
    out = _emb_lookup(flat_idx, type_embedding_weight)
    return out.reshape(token_types.shape + (D_MODEL,))
